# Initial kernel scaffold; baseline (speedup 1.0000x reference)
import jax
import jax.numpy as jnp
from jax.experimental import pallas as pl


def kernel(spixel_feats, index_map):
    raise NotImplementedError("write your pallas kernel here")


# speedup vs baseline: 8.7929x; 8.7929x over previous
import functools

import jax
import jax.numpy as jnp
from jax import lax
from jax.experimental import pallas as pl
from jax.experimental.pallas import tpu as pltpu
from jax.experimental.pallas import tpu_sc as plsc

B, C, K = 2, 96, 1024
H, W = 512, 512
P = H * W

NC, NS, L = 2, 16, 16
CH = C // NS
PCHUNK = 4096
NCHUNK = P // PCHUNK


_MESH = plsc.VectorSubcoreMesh(
    core_axis_name="c", subcore_axis_name="s", num_cores=NC, num_subcores=NS
)


@functools.partial(
    pl.kernel,
    out_type=jax.ShapeDtypeStruct((B, C, P), jnp.float32),
    mesh=_MESH,
    compiler_params=pltpu.CompilerParams(needs_layout_passes=False),
    scratch_types=[
        pltpu.VMEM((CH * K,), jnp.float32),
        pltpu.VMEM((PCHUNK,), jnp.int32),
        pltpu.VMEM((CH, PCHUNK), jnp.float32),
    ],
)
def _smear(feats_hbm, idx_hbm, out_hbm, table_v, idx_v, out_v):
    b = lax.axis_index("c")
    sid = lax.axis_index("s")
    c0 = sid * CH

    pltpu.sync_copy(feats_hbm.at[b, pl.ds(c0 * K, CH * K)], table_v)

    @pl.loop(0, NCHUNK)
    def _chunk(ch):
        base = ch * PCHUNK
        pltpu.sync_copy(idx_hbm.at[b, pl.ds(base, PCHUNK)], idx_v)

        @pl.loop(0, PCHUNK // L)
        def _vreg(v):
            iv = idx_v[pl.ds(v * L, L)]
            for j in range(CH):
                g = plsc.load_gather(table_v, [iv + (j * K)])
                out_v[j, pl.ds(v * L, L)] = g

        for j in range(CH):
            pltpu.sync_copy(
                out_v.at[j], out_hbm.at[b, c0 + j, pl.ds(base, PCHUNK)]
            )


def kernel(spixel_feats, index_map):
    feats2 = spixel_feats.reshape(B, C * K)
    idx2 = index_map.reshape(B, P).astype(jnp.int32)
    out = _smear(feats2, idx2)
    return out.reshape(B, C, H, W)


# speedup vs baseline: 27.2088x; 3.0944x over previous
import functools

import jax
import jax.numpy as jnp
from jax import lax
from jax.experimental import pallas as pl
from jax.experimental.pallas import tpu as pltpu
from jax.experimental.pallas import tpu_sc as plsc

B, C, K = 2, 96, 1024
H, W = 512, 512
P = H * W

NC, NS, L = 2, 16, 16
CH = C // NS
PCHUNK = 8192
NCHUNK = P // PCHUNK
NBUF = 2
NGRP = NCHUNK // NBUF
UNROLL = 8


_MESH = plsc.VectorSubcoreMesh(
    core_axis_name="c", subcore_axis_name="s", num_cores=NC, num_subcores=NS
)


@functools.partial(
    pl.kernel,
    out_type=jax.ShapeDtypeStruct((B, C, P), jnp.float32),
    mesh=_MESH,
    compiler_params=pltpu.CompilerParams(needs_layout_passes=False),
    scratch_types=[
        pltpu.VMEM((CH * K,), jnp.float32),
        pltpu.VMEM((PCHUNK,), jnp.int32),
        pltpu.VMEM((PCHUNK,), jnp.int32),
        pltpu.VMEM((CH * PCHUNK,), jnp.float32),
        pltpu.VMEM((CH * PCHUNK,), jnp.float32),
        pltpu.SemaphoreType.DMA,
        pltpu.SemaphoreType.DMA,
        pltpu.SemaphoreType.DMA,
        pltpu.SemaphoreType.DMA,
        pltpu.SemaphoreType.DMA,
    ],
)
def _smear(feats_hbm, idx_hbm, out_hbm, table_v, idx0, idx1, outb0, outb1,
           sem_t, si0, si1, so0, so1):
    b = lax.axis_index("c")
    sid = lax.axis_index("s")
    c0 = sid * CH

    idx_bufs = (idx0, idx1)
    out_bufs = (outb0, outb1)
    isems = (si0, si1)
    osems = (so0, so1)

    tbl_src = feats_hbm.at[b, pl.ds(c0 * K, CH * K)]
    pltpu.async_copy(tbl_src, table_v, sem_t)
    for u in range(NBUF):
        pltpu.async_copy(
            idx_hbm.at[b, pl.ds(u * PCHUNK, PCHUNK)], idx_bufs[u], isems[u]
        )
    pltpu.make_async_copy(tbl_src, table_v, sem_t).wait()

    @pl.loop(0, NGRP)
    def _grp(grp):
        for u in range(NBUF):
            ch = grp * NBUF + u
            base = ch * PCHUNK
            iv_ref, ov_ref = idx_bufs[u], out_bufs[u]
            isem, osem = isems[u], osems[u]

            pltpu.make_async_copy(
                idx_hbm.at[b, pl.ds(base, PCHUNK)], iv_ref, isem
            ).wait()

            @pl.when(grp > 0)
            def _drain():
                pbase = base - NBUF * PCHUNK
                for j in range(CH):
                    pltpu.make_async_copy(
                        ov_ref.at[pl.ds(j * PCHUNK, PCHUNK)],
                        out_hbm.at[b, c0 + j, pl.ds(pbase, PCHUNK)],
                        osem,
                    ).wait()

            @functools.partial(plsc.parallel_loop, 0, PCHUNK // L, unroll=UNROLL)
            def _vreg(v):
                iv = iv_ref[pl.ds(v * L, L)]
                for j in range(CH):
                    vals = plsc.load_gather(table_v, [iv + (j * K)])
                    ov_ref[pl.ds(j * PCHUNK + v * L, L)] = vals

            for j in range(CH):
                pltpu.async_copy(
                    ov_ref.at[pl.ds(j * PCHUNK, PCHUNK)],
                    out_hbm.at[b, c0 + j, pl.ds(base, PCHUNK)],
                    osem,
                )

            @pl.when(grp < NGRP - 1)
            def _prefetch():
                nbase = base + NBUF * PCHUNK
                pltpu.async_copy(
                    idx_hbm.at[b, pl.ds(nbase, PCHUNK)], iv_ref, isem
                )

    for u in range(NBUF):
        base = (NCHUNK - NBUF + u) * PCHUNK
        for j in range(CH):
            pltpu.make_async_copy(
                out_bufs[u].at[pl.ds(j * PCHUNK, PCHUNK)],
                out_hbm.at[b, c0 + j, pl.ds(base, PCHUNK)],
                osems[u],
            ).wait()


def kernel(spixel_feats, index_map):
    feats2 = spixel_feats.reshape(B, C * K)
    idx2 = index_map.reshape(B, P).astype(jnp.int32)
    out = _smear(feats2, idx2)
    return out.reshape(B, C, H, W)
